# baseline (speedup 1.0000x reference)
import functools

import jax
import jax.numpy as jnp
from jax import lax
from jax.experimental import pallas as pl
from jax.experimental.pallas import tpu as pltpu
from jax.experimental.pallas import tpu_sc as plsc

N, M, D, K, L = 20000, 4096, 64, 16, 3
R = 256
N_PAD = 20480
NB = N_PAD // R
HP = 136

SC_NC, SC_NS = 2, 16
SC_NW = SC_NC * SC_NS
SC_B = N_PAD * K
SC_PER_W = SC_B // SC_NW
SC_CH = 512
SC_NCHUNK = SC_PER_W // SC_CH
SC_SUB = 128
DG = 128


NCHUNK = M // 128
NLVL = 4


def _knn_body(x_ref, yt_ref, idx_ref, dist_ref, d2_ref):
    xb = x_ref[...]
    yt = yt_ref[...]
    xx = jnp.sum(xb * xb, axis=1, keepdims=True)
    yy = jnp.sum(yt * yt, axis=0, keepdims=True)
    d2_ref[...] = (
        xx - 2.0 * jnp.dot(xb, yt, preferred_element_type=jnp.float32) + yy)
    big = jnp.float32(jnp.inf)
    liota = lax.broadcasted_iota(jnp.int32, (8, 128), 1).astype(jnp.float32)

    def per_rows(s, carry):
        rows = pl.ds(s * 8, 8)
        w = [jnp.full((8, 128), big)] * NLVL
        cj = [jnp.zeros((8, 128))] * NLVL
        for j in range(NCHUNK):
            y = d2_ref[rows, pl.ds(j * 128, 128)]
            yj = jnp.full((8, 128), jnp.float32(j))
            for lv in range(NLVL):
                cond = y < w[lv]
                lo = jnp.minimum(y, w[lv])
                hi = jnp.maximum(y, w[lv])
                lo_j = jnp.where(cond, yj, cj[lv])
                yj = jnp.where(cond, cj[lv], yj)
                w[lv], cj[lv], y = lo, lo_j, hi
        for k in range(K):
            m = jnp.min(w[0], axis=1, keepdims=True)
            colreg = cj[0] * jnp.float32(128.0) + liota
            cand = jnp.where(w[0] == m, colreg, jnp.float32(1e9))
            col = jnp.min(cand, axis=1, keepdims=True)
            idx_ref[rows, k:k + 1] = col.astype(jnp.int32)
            dist_ref[rows, k:k + 1] = m
            popm = colreg == col
            for lv in range(NLVL - 1):
                w[lv] = jnp.where(popm, w[lv + 1], w[lv])
                cj[lv] = jnp.where(popm, cj[lv + 1], cj[lv])
            w[NLVL - 1] = jnp.where(popm, big, w[NLVL - 1])
        return carry

    lax.fori_loop(0, R // 8, per_rows, 0)


def _knn_call(x_pad, y_t):
    return pl.pallas_call(
        _knn_body,
        grid=(NB,),
        in_specs=[
            pl.BlockSpec((R, 8), lambda i: (i, 0)),
            pl.BlockSpec((8, M), lambda i: (0, 0)),
        ],
        out_specs=[
            pl.BlockSpec((R, K), lambda i: (i, 0)),
            pl.BlockSpec((R, K), lambda i: (i, 0)),
        ],
        out_shape=[
            jax.ShapeDtypeStruct((N_PAD, K), jnp.int32),
            jax.ShapeDtypeStruct((N_PAD, K), jnp.float32),
        ],
        scratch_shapes=[pltpu.VMEM((R, M), jnp.float32)],
    )(x_pad, y_t)


def _sc_gather_body(table_hbm, idx_hbm, out_hbm, idx_v, rows_v, sem):
    wid = lax.axis_index("s") * SC_NC + lax.axis_index("c")
    base = wid * SC_PER_W
    for c in range(SC_NCHUNK):
        off = base + c * SC_CH
        pltpu.sync_copy(idx_hbm.at[pl.ds(off, SC_CH)], idx_v)
        for j in range(SC_CH // SC_SUB):
            pltpu.async_copy(
                table_hbm.at[idx_v.at[pl.ds(j * SC_SUB, SC_SUB)]],
                rows_v.at[pl.ds(j * SC_SUB, SC_SUB), :],
                sem,
            )
        for j in range(SC_CH // SC_SUB):
            pltpu.make_async_copy(
                table_hbm.at[idx_v.at[pl.ds(j * SC_SUB, SC_SUB)]],
                rows_v.at[pl.ds(j * SC_SUB, SC_SUB), :],
                sem,
            ).wait()
        pltpu.sync_copy(rows_v, out_hbm.at[pl.ds(off, SC_CH)])


@functools.cache
def _make_sc_gather():
    return functools.partial(
        pl.kernel,
        mesh=plsc.VectorSubcoreMesh(core_axis_name="c", subcore_axis_name="s"),
        out_type=jax.ShapeDtypeStruct((SC_B, DG), jnp.float32),
        scratch_types=[
            pltpu.VMEM((SC_CH,), jnp.int32),
            pltpu.VMEM((SC_CH, DG), jnp.float32),
            pltpu.SemaphoreType.DMA,
        ],
    )(_sc_gather_body)


def _leaky(v):
    return jnp.where(v >= 0, v, 0.2 * v)


def _layers_body(g_ref, d_ref, w1_ref, w1y_ref, w2_ref, b1_ref, b2_ref,
                 gw_ref, gb_ref, out_ref):
    emb = jnp.ones((R, D), dtype=jnp.float32)
    liota = lax.broadcasted_iota(jnp.int32, (R, D), 1)
    mask_lo = liota < (D // 2)
    for l in range(L):
        w1p = w1_ref[l, 0:D, :]
        w1y = w1y_ref[l]
        w1d = w1_ref[l, 2 * D:2 * D + 1, :]
        b1 = b1_ref[l, 0:1, :]
        p = jnp.dot(emb, w1p, preferred_element_type=jnp.float32) + b1
        acc = jnp.zeros((R, HP), dtype=jnp.float32)
        for k in range(K):
            gk = g_ref[k]
            hk = jnp.dot(gk, w1y, preferred_element_type=jnp.float32)
            hk = hk + d_ref[k] * w1d + p
            acc = acc + _leaky(hk)
        msg = jnp.dot(acc, w2_ref[l], preferred_element_type=jnp.float32)
        msg = msg + jnp.float32(K) * b2_ref[l, 0:1, :]
        s_lo = jnp.sum(jnp.where(mask_lo, msg, 0.0), axis=1, keepdims=True)
        s_hi = jnp.sum(jnp.where(mask_lo, 0.0, msg), axis=1, keepdims=True)
        mu = jnp.where(mask_lo, s_lo, s_hi) * jnp.float32(2.0 / D)
        diff = msg - mu
        v_lo = jnp.sum(jnp.where(mask_lo, diff * diff, 0.0), axis=1,
                       keepdims=True)
        v_hi = jnp.sum(jnp.where(mask_lo, 0.0, diff * diff), axis=1,
                       keepdims=True)
        var = jnp.where(mask_lo, v_lo, v_hi) * jnp.float32(2.0 / D)
        hn = diff * lax.rsqrt(var + jnp.float32(1e-5))
        gn = hn * gw_ref[l, 0:1, :] + gb_ref[l, 0:1, :]
        emb = emb + _leaky(gn)
    out_ref[...] = emb


def _layers_call(g3, d3, w1s, w1ys, w2s, b1s, b2s, gws, gbs):
    return pl.pallas_call(
        _layers_body,
        grid=(NB,),
        in_specs=[
            pl.BlockSpec((K, R, DG), lambda i: (0, i, 0)),
            pl.BlockSpec((K, R, 1), lambda i: (0, i, 0)),
            pl.BlockSpec((L, HP, HP), lambda i: (0, 0, 0)),
            pl.BlockSpec((L, DG, HP), lambda i: (0, 0, 0)),
            pl.BlockSpec((L, HP, D), lambda i: (0, 0, 0)),
            pl.BlockSpec((L, 8, HP), lambda i: (0, 0, 0)),
            pl.BlockSpec((L, 8, D), lambda i: (0, 0, 0)),
            pl.BlockSpec((L, 8, D), lambda i: (0, 0, 0)),
            pl.BlockSpec((L, 8, D), lambda i: (0, 0, 0)),
        ],
        out_specs=pl.BlockSpec((R, D), lambda i: (i, 0)),
        out_shape=jax.ShapeDtypeStruct((N_PAD, D), jnp.float32),
    )(g3, d3, w1s, w1ys, w2s, b1s, b2s, gws, gbs)


def _pack_params(W1s, b1s, W2s, b2s, gws, gbs):
    h = 2 * D + 1
    w1 = jnp.stack([jnp.pad(w, ((0, HP - h), (0, HP - h))) for w in W1s])
    w1y = jnp.stack([jnp.pad(w[D:2 * D, :], ((0, DG - D), (0, HP - h)))
                     for w in W1s])
    w2 = jnp.stack([jnp.pad(w, ((0, HP - h), (0, 0))) for w in W2s])
    b1 = jnp.stack([jnp.pad(b[None, :], ((0, 7), (0, HP - h))) for b in b1s])
    b2 = jnp.stack([jnp.pad(b[None, :], ((0, 7), (0, 0))) for b in b2s])
    gw = jnp.stack([jnp.pad(g[None, :], ((0, 7), (0, 0))) for g in gws])
    gb = jnp.stack([jnp.pad(g[None, :], ((0, 7), (0, 0))) for g in gbs])
    return w1, w1y, w2, b1, b2, gw, gb


def kernel(x, y, y_atomtypes, x_batch, y_batch,
           W1_0, b1_0, W2_0, b2_0, gn_w_0, gn_b_0,
           W1_1, b1_1, W2_1, b2_1, gn_w_1, gn_b_1,
           W1_2, b1_2, W2_2, b2_2, gn_w_2, gn_b_2):
    x_pad = jnp.pad(x, ((0, N_PAD - N), (0, 5)))
    y_t = jnp.pad(y, ((0, 0), (0, 5))).T
    idx, dists = _knn_call(x_pad, y_t)

    idx_t = idx.T.reshape(-1)
    table = jnp.pad(y_atomtypes, ((0, 0), (0, DG - D)))
    g = _make_sc_gather()(table, idx_t)
    g3 = g.reshape(K, N_PAD, DG)
    d3 = dists.T.reshape(K, N_PAD, 1)

    w1, w1y, w2, b1, b2, gw, gb = _pack_params(
        (W1_0, W1_1, W1_2), (b1_0, b1_1, b1_2),
        (W2_0, W2_1, W2_2), (b2_0, b2_1, b2_2),
        (gn_w_0, gn_w_1, gn_w_2), (gn_b_0, gn_b_1, gn_b_2))
    emb = _layers_call(g3, d3, w1, w1y, w2, b1, b2, gw, gb)
    return emb[:N]


# speedup vs baseline: 6.3906x; 6.3906x over previous
import functools

import jax
import jax.numpy as jnp
from jax import lax
from jax.experimental import pallas as pl
from jax.experimental.pallas import tpu as pltpu
from jax.experimental.pallas import tpu_sc as plsc

N, M, D, K, L = 20000, 4096, 64, 16, 3
R = 256
N_PAD = 20480
NB = N_PAD // R
HP = 136

SC_NC, SC_NS = 2, 16
SC_NW = SC_NC * SC_NS
SC_B = N_PAD * K
SC_PER_W = SC_B // SC_NW
SC_CH = 512
SC_NCHUNK = SC_PER_W // SC_CH
SC_SUB = 128
DG = 128


NCHUNK = M // 128
NLVL = 4


def _knn_body(x_ref, yt_ref, idx_ref, dist_ref, d2_ref):
    xb = x_ref[...]
    yt = yt_ref[...]
    xx = jnp.sum(xb * xb, axis=1, keepdims=True)
    yy = jnp.sum(yt * yt, axis=0, keepdims=True)
    d2_ref[...] = (
        xx - 2.0 * jnp.dot(xb, yt, preferred_element_type=jnp.float32) + yy)
    big = jnp.float32(jnp.inf)
    RG = 32
    liota = lax.broadcasted_iota(jnp.int32, (RG, 128), 1).astype(jnp.float32)

    for s in range(R // RG):
        r0 = s * RG
        w = [jnp.full((RG, 128), big)] * NLVL
        cj = [jnp.zeros((RG, 128))] * NLVL
        for j in range(NCHUNK):
            y = d2_ref[r0:r0 + RG, j * 128:(j + 1) * 128]
            yj = jnp.full((RG, 128), jnp.float32(j))
            for lv in range(NLVL):
                cond = y < w[lv]
                lo = jnp.minimum(y, w[lv])
                hi = jnp.maximum(y, w[lv])
                lo_j = jnp.where(cond, yj, cj[lv])
                yj = jnp.where(cond, cj[lv], yj)
                w[lv], cj[lv], y = lo, lo_j, hi
        for k in range(K):
            m = jnp.min(w[0], axis=1, keepdims=True)
            colreg = cj[0] * jnp.float32(128.0) + liota
            cand = jnp.where(w[0] == m, colreg, jnp.float32(1e9))
            col = jnp.min(cand, axis=1, keepdims=True)
            idx_ref[r0:r0 + RG, k:k + 1] = col.astype(jnp.int32)
            dist_ref[r0:r0 + RG, k:k + 1] = m
            popm = colreg == col
            for lv in range(NLVL - 1):
                w[lv] = jnp.where(popm, w[lv + 1], w[lv])
                cj[lv] = jnp.where(popm, cj[lv + 1], cj[lv])
            w[NLVL - 1] = jnp.where(popm, big, w[NLVL - 1])


def _knn_call(x_pad, y_t):
    return pl.pallas_call(
        _knn_body,
        grid=(NB,),
        in_specs=[
            pl.BlockSpec((R, 8), lambda i: (i, 0)),
            pl.BlockSpec((8, M), lambda i: (0, 0)),
        ],
        out_specs=[
            pl.BlockSpec((R, K), lambda i: (i, 0)),
            pl.BlockSpec((R, K), lambda i: (i, 0)),
        ],
        out_shape=[
            jax.ShapeDtypeStruct((N_PAD, K), jnp.int32),
            jax.ShapeDtypeStruct((N_PAD, K), jnp.float32),
        ],
        scratch_shapes=[pltpu.VMEM((R, M), jnp.float32)],
    )(x_pad, y_t)


def _sc_gather_body(table_hbm, idx_hbm, out_hbm, idx_v, rows_v, sem):
    wid = lax.axis_index("s") * SC_NC + lax.axis_index("c")
    base = wid * SC_PER_W
    for c in range(SC_NCHUNK):
        off = base + c * SC_CH
        pltpu.sync_copy(idx_hbm.at[pl.ds(off, SC_CH)], idx_v)
        for j in range(SC_CH // SC_SUB):
            pltpu.async_copy(
                table_hbm.at[idx_v.at[pl.ds(j * SC_SUB, SC_SUB)]],
                rows_v.at[pl.ds(j * SC_SUB, SC_SUB), :],
                sem,
            )
        for j in range(SC_CH // SC_SUB):
            pltpu.make_async_copy(
                table_hbm.at[idx_v.at[pl.ds(j * SC_SUB, SC_SUB)]],
                rows_v.at[pl.ds(j * SC_SUB, SC_SUB), :],
                sem,
            ).wait()
        pltpu.sync_copy(rows_v, out_hbm.at[pl.ds(off, SC_CH)])


@functools.cache
def _make_sc_gather():
    return functools.partial(
        pl.kernel,
        mesh=plsc.VectorSubcoreMesh(core_axis_name="c", subcore_axis_name="s"),
        out_type=jax.ShapeDtypeStruct((SC_B, DG), jnp.float32),
        scratch_types=[
            pltpu.VMEM((SC_CH,), jnp.int32),
            pltpu.VMEM((SC_CH, DG), jnp.float32),
            pltpu.SemaphoreType.DMA,
        ],
    )(_sc_gather_body)


def _leaky(v):
    return jnp.where(v >= 0, v, 0.2 * v)


def _layers_body(g_ref, d_ref, w1_ref, w1y_ref, w2_ref, b1_ref, b2_ref,
                 gw_ref, gb_ref, out_ref):
    emb = jnp.ones((R, D), dtype=jnp.float32)
    liota = lax.broadcasted_iota(jnp.int32, (R, D), 1)
    mask_lo = liota < (D // 2)
    for l in range(L):
        w1p = w1_ref[l, 0:D, :]
        w1y = w1y_ref[l]
        w1d = w1_ref[l, 2 * D:2 * D + 1, :]
        b1 = b1_ref[l, 0:1, :]
        p = jnp.dot(emb, w1p, preferred_element_type=jnp.float32) + b1
        acc = jnp.zeros((R, HP), dtype=jnp.float32)
        for k in range(K):
            gk = g_ref[k]
            hk = jnp.dot(gk, w1y, preferred_element_type=jnp.float32)
            hk = hk + d_ref[k] * w1d + p
            acc = acc + _leaky(hk)
        msg = jnp.dot(acc, w2_ref[l], preferred_element_type=jnp.float32)
        msg = msg + jnp.float32(K) * b2_ref[l, 0:1, :]
        s_lo = jnp.sum(jnp.where(mask_lo, msg, 0.0), axis=1, keepdims=True)
        s_hi = jnp.sum(jnp.where(mask_lo, 0.0, msg), axis=1, keepdims=True)
        mu = jnp.where(mask_lo, s_lo, s_hi) * jnp.float32(2.0 / D)
        diff = msg - mu
        v_lo = jnp.sum(jnp.where(mask_lo, diff * diff, 0.0), axis=1,
                       keepdims=True)
        v_hi = jnp.sum(jnp.where(mask_lo, 0.0, diff * diff), axis=1,
                       keepdims=True)
        var = jnp.where(mask_lo, v_lo, v_hi) * jnp.float32(2.0 / D)
        hn = diff * lax.rsqrt(var + jnp.float32(1e-5))
        gn = hn * gw_ref[l, 0:1, :] + gb_ref[l, 0:1, :]
        emb = emb + _leaky(gn)
    out_ref[...] = emb


def _layers_call(g3, d3, w1s, w1ys, w2s, b1s, b2s, gws, gbs):
    return pl.pallas_call(
        _layers_body,
        grid=(NB,),
        in_specs=[
            pl.BlockSpec((K, R, DG), lambda i: (0, i, 0)),
            pl.BlockSpec((K, R, 1), lambda i: (0, i, 0)),
            pl.BlockSpec((L, HP, HP), lambda i: (0, 0, 0)),
            pl.BlockSpec((L, DG, HP), lambda i: (0, 0, 0)),
            pl.BlockSpec((L, HP, D), lambda i: (0, 0, 0)),
            pl.BlockSpec((L, 8, HP), lambda i: (0, 0, 0)),
            pl.BlockSpec((L, 8, D), lambda i: (0, 0, 0)),
            pl.BlockSpec((L, 8, D), lambda i: (0, 0, 0)),
            pl.BlockSpec((L, 8, D), lambda i: (0, 0, 0)),
        ],
        out_specs=pl.BlockSpec((R, D), lambda i: (i, 0)),
        out_shape=jax.ShapeDtypeStruct((N_PAD, D), jnp.float32),
    )(g3, d3, w1s, w1ys, w2s, b1s, b2s, gws, gbs)


def _pack_params(W1s, b1s, W2s, b2s, gws, gbs):
    h = 2 * D + 1
    w1 = jnp.stack([jnp.pad(w, ((0, HP - h), (0, HP - h))) for w in W1s])
    w1y = jnp.stack([jnp.pad(w[D:2 * D, :], ((0, DG - D), (0, HP - h)))
                     for w in W1s])
    w2 = jnp.stack([jnp.pad(w, ((0, HP - h), (0, 0))) for w in W2s])
    b1 = jnp.stack([jnp.pad(b[None, :], ((0, 7), (0, HP - h))) for b in b1s])
    b2 = jnp.stack([jnp.pad(b[None, :], ((0, 7), (0, 0))) for b in b2s])
    gw = jnp.stack([jnp.pad(g[None, :], ((0, 7), (0, 0))) for g in gws])
    gb = jnp.stack([jnp.pad(g[None, :], ((0, 7), (0, 0))) for g in gbs])
    return w1, w1y, w2, b1, b2, gw, gb


def kernel(x, y, y_atomtypes, x_batch, y_batch,
           W1_0, b1_0, W2_0, b2_0, gn_w_0, gn_b_0,
           W1_1, b1_1, W2_1, b2_1, gn_w_1, gn_b_1,
           W1_2, b1_2, W2_2, b2_2, gn_w_2, gn_b_2):
    x_pad = jnp.pad(x, ((0, N_PAD - N), (0, 5)))
    y_t = jnp.pad(y, ((0, 0), (0, 5))).T
    idx, dists = _knn_call(x_pad, y_t)

    idx_t = idx.T.reshape(-1)
    table = jnp.pad(y_atomtypes, ((0, 0), (0, DG - D)))
    g = _make_sc_gather()(table, idx_t)
    g3 = g.reshape(K, N_PAD, DG)
    d3 = dists.T.reshape(K, N_PAD, 1)

    w1, w1y, w2, b1, b2, gw, gb = _pack_params(
        (W1_0, W1_1, W1_2), (b1_0, b1_1, b1_2),
        (W2_0, W2_1, W2_2), (b2_0, b2_1, b2_2),
        (gn_w_0, gn_w_1, gn_w_2), (gn_b_0, gn_b_1, gn_b_2))
    emb = _layers_call(g3, d3, w1, w1y, w2, b1, b2, gw, gb)
    return emb[:N]


# speedup vs baseline: 6.8818x; 1.0769x over previous
import functools

import jax
import jax.numpy as jnp
from jax import lax
from jax.experimental import pallas as pl
from jax.experimental.pallas import tpu as pltpu
from jax.experimental.pallas import tpu_sc as plsc

N, M, D, K, L = 20000, 4096, 64, 16, 3
R = 256
N_PAD = 20480
NB = N_PAD // R
HP = 136

SC_NC, SC_NS = 2, 16
SC_NW = SC_NC * SC_NS
SC_B = N_PAD * K
SC_PER_W = SC_B // SC_NW
SC_CH = 512
SC_NCHUNK = SC_PER_W // SC_CH
SC_SUB = 128
DG = 128


NCHUNK = M // 128
NLVL = 4


def _knn_body(x_ref, yt_ref, idx_ref, dist_ref, d2_ref):
    xb = x_ref[...]
    yt = yt_ref[...]
    xx = jnp.sum(xb * xb, axis=1, keepdims=True)
    yy = jnp.sum(yt * yt, axis=0, keepdims=True)
    d2_ref[...] = (
        xx - 2.0 * jnp.dot(xb, yt, preferred_element_type=jnp.float32) + yy)
    big = jnp.float32(jnp.inf)
    RG = 32
    liota = lax.broadcasted_iota(jnp.int32, (RG, 128), 1).astype(jnp.float32)

    for s in range(R // RG):
        r0 = s * RG
        w = [jnp.full((RG, 128), big)] * NLVL
        cj = [jnp.zeros((RG, 128))] * NLVL
        for j in range(NCHUNK):
            y = d2_ref[r0:r0 + RG, j * 128:(j + 1) * 128]
            yj = jnp.full((RG, 128), jnp.float32(j))
            for lv in range(NLVL):
                cond = y < w[lv]
                lo = jnp.minimum(y, w[lv])
                hi = jnp.maximum(y, w[lv])
                lo_j = jnp.where(cond, yj, cj[lv])
                yj = jnp.where(cond, cj[lv], yj)
                w[lv], cj[lv], y = lo, lo_j, hi
        for k in range(K):
            m = jnp.min(w[0], axis=1, keepdims=True)
            colreg = cj[0] * jnp.float32(128.0) + liota
            cand = jnp.where(w[0] == m, colreg, jnp.float32(1e9))
            col = jnp.min(cand, axis=1, keepdims=True)
            idx_ref[r0:r0 + RG, k:k + 1] = col.astype(jnp.int32)
            dist_ref[r0:r0 + RG, k:k + 1] = m
            popm = colreg == col
            for lv in range(NLVL - 1):
                w[lv] = jnp.where(popm, w[lv + 1], w[lv])
                cj[lv] = jnp.where(popm, cj[lv + 1], cj[lv])
            w[NLVL - 1] = jnp.where(popm, big, w[NLVL - 1])


def _knn_call(x_pad, y_t, nrows):
    return pl.pallas_call(
        _knn_body,
        grid=(nrows // R,),
        in_specs=[
            pl.BlockSpec((R, 8), lambda i: (i, 0)),
            pl.BlockSpec((8, M), lambda i: (0, 0)),
        ],
        out_specs=[
            pl.BlockSpec((R, K), lambda i: (i, 0)),
            pl.BlockSpec((R, K), lambda i: (i, 0)),
        ],
        out_shape=[
            jax.ShapeDtypeStruct((nrows, K), jnp.int32),
            jax.ShapeDtypeStruct((nrows, K), jnp.float32),
        ],
        scratch_shapes=[pltpu.VMEM((R, M), jnp.float32)],
    )(x_pad, y_t)


def _sc_gather_body(per_w, table_hbm, idx_hbm, out_hbm, idx_v, rows_v, sem):
    wid = lax.axis_index("s") * SC_NC + lax.axis_index("c")
    base = wid * per_w
    for c in range(per_w // SC_CH):
        off = base + c * SC_CH
        pltpu.sync_copy(idx_hbm.at[pl.ds(off, SC_CH)], idx_v)
        for j in range(SC_CH // SC_SUB):
            pltpu.async_copy(
                table_hbm.at[idx_v.at[pl.ds(j * SC_SUB, SC_SUB)]],
                rows_v.at[pl.ds(j * SC_SUB, SC_SUB), :],
                sem,
            )
        for j in range(SC_CH // SC_SUB):
            pltpu.make_async_copy(
                table_hbm.at[idx_v.at[pl.ds(j * SC_SUB, SC_SUB)]],
                rows_v.at[pl.ds(j * SC_SUB, SC_SUB), :],
                sem,
            ).wait()
        pltpu.sync_copy(rows_v, out_hbm.at[pl.ds(off, SC_CH)])


@functools.cache
def _make_sc_gather(nb):
    return functools.partial(
        pl.kernel,
        mesh=plsc.VectorSubcoreMesh(core_axis_name="c", subcore_axis_name="s"),
        out_type=jax.ShapeDtypeStruct((nb, DG), jnp.float32),
        scratch_types=[
            pltpu.VMEM((SC_CH,), jnp.int32),
            pltpu.VMEM((SC_CH, DG), jnp.float32),
            pltpu.SemaphoreType.DMA,
        ],
    )(functools.partial(_sc_gather_body, nb // SC_NW))


def _leaky(v):
    return jnp.maximum(v, 0.2 * v)


def _layers_body(g_ref, d_ref, w1_ref, w1y_ref, w2_ref, b1_ref, b2_ref,
                 gw_ref, gb_ref, out_ref):
    emb = jnp.ones((R, D), dtype=jnp.float32)
    liota = lax.broadcasted_iota(jnp.int32, (R, D), 1)
    mask_lo = liota < (D // 2)
    for l in range(L):
        w1p = w1_ref[l, 0:D, :]
        w1y = w1y_ref[l]
        w1d = w1_ref[l, 2 * D:2 * D + 1, :]
        b1 = b1_ref[l, 0:1, :]
        p = jnp.dot(emb, w1p, preferred_element_type=jnp.float32) + b1
        acc = jnp.zeros((R, HP), dtype=jnp.float32)
        for k in range(K):
            gk = g_ref[k]
            hk = jnp.dot(gk, w1y, preferred_element_type=jnp.float32)
            hk = hk + (d_ref[k] * w1d + p)
            acc = acc + _leaky(hk)
        msg = jnp.dot(acc, w2_ref[l], preferred_element_type=jnp.float32)
        msg = msg + jnp.float32(K) * b2_ref[l, 0:1, :]
        s_lo = jnp.sum(jnp.where(mask_lo, msg, 0.0), axis=1, keepdims=True)
        s_hi = jnp.sum(jnp.where(mask_lo, 0.0, msg), axis=1, keepdims=True)
        mu = jnp.where(mask_lo, s_lo, s_hi) * jnp.float32(2.0 / D)
        diff = msg - mu
        v_lo = jnp.sum(jnp.where(mask_lo, diff * diff, 0.0), axis=1,
                       keepdims=True)
        v_hi = jnp.sum(jnp.where(mask_lo, 0.0, diff * diff), axis=1,
                       keepdims=True)
        var = jnp.where(mask_lo, v_lo, v_hi) * jnp.float32(2.0 / D)
        hn = diff * lax.rsqrt(var + jnp.float32(1e-5))
        gn = hn * gw_ref[l, 0:1, :] + gb_ref[l, 0:1, :]
        emb = emb + _leaky(gn)
    out_ref[...] = emb


def _layers_call(g3, d3, w1s, w1ys, w2s, b1s, b2s, gws, gbs, nrows):
    return pl.pallas_call(
        _layers_body,
        grid=(nrows // R,),
        in_specs=[
            pl.BlockSpec((K, R, DG), lambda i: (0, i, 0)),
            pl.BlockSpec((K, R, 1), lambda i: (0, i, 0)),
            pl.BlockSpec((L, HP, HP), lambda i: (0, 0, 0)),
            pl.BlockSpec((L, DG, HP), lambda i: (0, 0, 0)),
            pl.BlockSpec((L, HP, D), lambda i: (0, 0, 0)),
            pl.BlockSpec((L, 8, HP), lambda i: (0, 0, 0)),
            pl.BlockSpec((L, 8, D), lambda i: (0, 0, 0)),
            pl.BlockSpec((L, 8, D), lambda i: (0, 0, 0)),
            pl.BlockSpec((L, 8, D), lambda i: (0, 0, 0)),
        ],
        out_specs=pl.BlockSpec((R, D), lambda i: (i, 0)),
        out_shape=jax.ShapeDtypeStruct((nrows, D), jnp.float32),
    )(g3, d3, w1s, w1ys, w2s, b1s, b2s, gws, gbs)


def _pack_params(W1s, b1s, W2s, b2s, gws, gbs):
    h = 2 * D + 1
    w1 = jnp.stack([jnp.pad(w, ((0, HP - h), (0, HP - h))) for w in W1s])
    w1y = jnp.stack([jnp.pad(w[D:2 * D, :], ((0, DG - D), (0, HP - h)))
                     for w in W1s])
    w2 = jnp.stack([jnp.pad(w, ((0, HP - h), (0, 0))) for w in W2s])
    b1 = jnp.stack([jnp.pad(b[None, :], ((0, 7), (0, HP - h))) for b in b1s])
    b2 = jnp.stack([jnp.pad(b[None, :], ((0, 7), (0, 0))) for b in b2s])
    gw = jnp.stack([jnp.pad(g[None, :], ((0, 7), (0, 0))) for g in gws])
    gb = jnp.stack([jnp.pad(g[None, :], ((0, 7), (0, 0))) for g in gbs])
    return w1, w1y, w2, b1, b2, gw, gb


def kernel(x, y, y_atomtypes, x_batch, y_batch,
           W1_0, b1_0, W2_0, b2_0, gn_w_0, gn_b_0,
           W1_1, b1_1, W2_1, b2_1, gn_w_1, gn_b_1,
           W1_2, b1_2, W2_2, b2_2, gn_w_2, gn_b_2):
    x_pad = jnp.pad(x, ((0, N_PAD - N), (0, 5)))
    y_t = jnp.pad(y, ((0, 0), (0, 5))).T
    table = jnp.pad(y_atomtypes, ((0, 0), (0, DG - D)))
    w1, w1y, w2, b1, b2, gw, gb = _pack_params(
        (W1_0, W1_1, W1_2), (b1_0, b1_1, b1_2),
        (W2_0, W2_1, W2_2), (b2_0, b2_1, b2_2),
        (gn_w_0, gn_w_1, gn_w_2), (gn_b_0, gn_b_1, gn_b_2))

    H = N_PAD // 2
    embs = []
    for x_h in (x_pad[:H], x_pad[H:]):
        idx, dists = _knn_call(x_h, y_t, H)
        idx_t = idx.T.reshape(-1)
        g = _make_sc_gather(K * H)(table, idx_t)
        g3 = g.reshape(K, H, DG)
        d3 = dists.T.reshape(K, H, 1)
        embs.append(_layers_call(g3, d3, w1, w1y, w2, b1, b2, gw, gb, H))
    return jnp.concatenate(embs, axis=0)[:N]
